# baseline (speedup 1.0000x reference)
import functools

import jax
import jax.numpy as jnp
from jax.experimental import pallas as pl
from jax.experimental.pallas import tpu as pltpu

_HS = 768
_E = 64
_TOPK = 8
_BT = 4096


def _router_body(x_ref, w_ref, wout_ref, iout_ref):
    x = x_ref[...]
    w = w_ref[...]
    logits = jax.lax.dot_general(
        w, x, (((1,), (1,)), ((), ())),
        preferred_element_type=jnp.float32)
    m = jnp.max(logits, axis=0, keepdims=True)
    p = jnp.exp(logits - m)
    rdenom = 1.0 / jnp.sum(p, axis=0, keepdims=True)

    eidx = jax.lax.broadcasted_iota(jnp.int32, (_E, _BT), 0)
    vals = p
    wrows = []
    irows = []
    for k in range(_TOPK):
        mk = jnp.max(vals, axis=0, keepdims=True)
        hit = vals == mk
        idx = jnp.min(jnp.where(hit, eidx, _E), axis=0, keepdims=True)
        wrows.append(mk * rdenom)
        irows.append(idx)
        vals = jnp.where(eidx == idx, -1.0, vals)
    wout_ref[...] = jnp.concatenate(wrows, axis=0).T
    iout_ref[...] = jnp.concatenate(irows, axis=0).T


@jax.jit
def _router(xf, W):
    n = xf.shape[0]
    grid = (n // _BT,)
    return pl.pallas_call(
        _router_body,
        grid=grid,
        in_specs=[
            pl.BlockSpec((_BT, _HS), lambda i: (i, 0)),
            pl.BlockSpec((_E, _HS), lambda i: (0, 0)),
        ],
        out_specs=[
            pl.BlockSpec((_BT, _TOPK), lambda i: (i, 0)),
            pl.BlockSpec((_BT, _TOPK), lambda i: (i, 0)),
        ],
        out_shape=[
            jax.ShapeDtypeStruct((n, _TOPK), jnp.float32),
            jax.ShapeDtypeStruct((n, _TOPK), jnp.int32),
        ],
    )(xf, W)


def kernel(x, W):
    xf = x.reshape(-1, x.shape[-1])
    return _router(xf, W)
